# Initial kernel scaffold; baseline (speedup 1.0000x reference)
import jax
import jax.numpy as jnp
from jax.experimental import pallas as pl


def kernel(pred_lines, confidence, targets):
    raise NotImplementedError("write your pallas kernel here")


# speedup vs baseline: 9.0480x; 9.0480x over previous
import functools

import jax
import jax.numpy as jnp
from jax import lax
from jax.experimental import pallas as pl
from jax.experimental.pallas import tpu as pltpu
from jax.experimental.pallas import tpu_sc as plsc

_NC, _NS = 2, 16
_NW = _NC * _NS
_B, _K, _T = 64, 1000, 100
_KP, _TP = 1024, 128
_NCHUNK = 1008 // 16
_BPW = _B // _NW
_BIG = 1e30


def _rsqrt_newton(x):
    i = lax.bitcast_convert_type(x, jnp.int32)
    y = lax.bitcast_convert_type(
        jnp.int32(0x5F3759DF) - lax.shift_right_arithmetic(i, 1), jnp.float32)
    for _ in range(3):
        y = y * (1.5 - 0.5 * x * y * y)
    return y


def _sc_match(p0, p1, p2, nsig, conf, t0, t1, t2):
    mesh = plsc.VectorSubcoreMesh(core_axis_name="c", subcore_axis_name="s")
    fkp = lambda: pltpu.VMEM((_KP,), jnp.float32)
    ftp = lambda: pltpu.VMEM((_TP,), jnp.float32)

    @functools.partial(
        pl.kernel,
        out_type=jax.ShapeDtypeStruct((_B, 16), jnp.float32),
        mesh=mesh,
        scratch_types=[fkp(), fkp(), fkp(), fkp(), fkp(),
                       ftp(), ftp(), ftp(), ftp(), ftp(), ftp(),
                       pltpu.VMEM((16,), jnp.float32)],
        compiler_params=pltpu.CompilerParams(needs_layout_passes=False),
    )
    def k(p0_h, p1_h, p2_h, nsig_h, conf_h, t0_h, t1_h, t2_h, out_h,
          p0v, p1v, p2v, basev, confv, t0v, t1v, t2v, m0v, m1v, m2v, rowv):
        wid = lax.axis_index("s") * _NC + lax.axis_index("c")
        iota = lax.iota(jnp.int32, 16)
        for r in range(_BPW):
            b = wid * _BPW + r
            pltpu.sync_copy(p0_h.at[b], p0v)
            pltpu.sync_copy(p1_h.at[b], p1v)
            pltpu.sync_copy(p2_h.at[b], p2v)
            pltpu.sync_copy(nsig_h.at[b], basev)
            pltpu.sync_copy(conf_h.at[b], confv)
            pltpu.sync_copy(t0_h.at[b], t0v)
            pltpu.sync_copy(t1_h.at[b], t1v)
            pltpu.sync_copy(t2_h.at[b], t2v)

            lane0 = iota == 0

            def jstep(j, cm):
                j_s = jnp.full((16,), j, jnp.int32)
                t0s = plsc.load_gather(t0v, (j_s,))
                t1s = plsc.load_gather(t1v, (j_s,))
                t2s = plsc.load_gather(t2v, (j_s,))

                def cstep(c, carry):
                    rm, ri = carry
                    off = c * 16
                    v = (jnp.abs(p0v[pl.ds(off, 16)] - t0s)
                         + jnp.abs(p1v[pl.ds(off, 16)] - t1s)
                         + jnp.abs(p2v[pl.ds(off, 16)] - t2s)
                         ) + basev[pl.ds(off, 16)]
                    better = v < rm
                    rm = jnp.where(better, v, rm)
                    ri = jnp.where(better, off + iota, ri)
                    return rm, ri

                rm0 = jnp.full((16,), 3e38, jnp.float32)
                ri0 = jnp.zeros((16,), jnp.int32)
                rm, ri = lax.fori_loop(0, _NCHUNK, cstep, (rm0, ri0))
                mn = jnp.min(rm)
                idx = jnp.min(jnp.where(rm == mn, ri, jnp.int32(1 << 30)))
                idx_s = jnp.full((16,), idx, jnp.int32)
                plsc.store_scatter(basev, (idx_s,),
                                   jnp.full((16,), _BIG, jnp.float32),
                                   mask=lane0)
                plsc.store_scatter(m0v, (j_s,),
                                   plsc.load_gather(p0v, (idx_s,)), mask=lane0)
                plsc.store_scatter(m1v, (j_s,),
                                   plsc.load_gather(p1v, (idx_s,)), mask=lane0)
                plsc.store_scatter(m2v, (j_s,),
                                   plsc.load_gather(p2v, (idx_s,)), mask=lane0)
                return cm + plsc.load_gather(confv, (idx_s,))[0]

            cm = lax.fori_loop(0, _T, jstep, jnp.float32(0.0))

            dir_acc = jnp.zeros((16,), jnp.float32)
            off_acc = jnp.zeros((16,), jnp.float32)
            for cc in range((_T + 15) // 16):
                s = cc * 16
                a0 = m0v[pl.ds(s, 16)]
                a1 = m1v[pl.ds(s, 16)]
                ad = m2v[pl.ds(s, 16)]
                b0 = t0v[pl.ds(s, 16)]
                b1 = t1v[pl.ds(s, 16)]
                bd = t2v[pl.ds(s, 16)]
                sp = jnp.maximum(a0 * a0 + a1 * a1, 1e-24)
                st = jnp.maximum(b0 * b0 + b1 * b1, 1e-24)
                rp = _rsqrt_newton(sp)
                rt = _rsqrt_newton(st)
                u = (a0 * b0 + a1 * b1) * (rp * rt)
                dirt = jnp.abs(1.0 - u)
                offt = jnp.abs(ad * rp - bd * rt)
                msk = (s + iota) < _T
                dir_acc = dir_acc + jnp.where(msk, dirt, 0.0)
                off_acc = off_acc + jnp.where(msk, offt, 0.0)
            dir_s = jnp.sum(dir_acc)
            off_s = jnp.sum(off_acc)
            row = (jnp.where(iota == 0, dir_s, 0.0)
                   + jnp.where(iota == 1, off_s, 0.0)
                   + jnp.where(iota == 2, cm, 0.0))
            rowv[...] = row
            pltpu.sync_copy(rowv, out_h.at[b])

    return k(p0, p1, p2, nsig, conf, t0, t1, t2)


def _tc_combine(conf, partials):
    def body(conf_ref, part_ref, out_ref):
        x = conf_ref[...]
        bce = jnp.sum(jnp.maximum(x, 0.0) + jnp.log1p(jnp.exp(-jnp.abs(x))))
        pr = part_ref[...]
        dir_tot = jnp.sum(pr[:, 0:1])
        off_tot = jnp.sum(pr[:, 1:2])
        cm_tot = jnp.sum(pr[:, 2:3])
        inv_bt = 1.0 / (_B * _T)
        loss_lines = (dir_tot * inv_bt + off_tot * inv_bt) * 0.5
        loss_conf = (bce - cm_tot) * (1.0 / (_B * _K))
        out_ref[0, 0] = (loss_lines + loss_conf) * 0.5

    return pl.pallas_call(
        body,
        out_shape=jax.ShapeDtypeStruct((1, 1), jnp.float32),
        out_specs=pl.BlockSpec(memory_space=pltpu.SMEM),
    )(conf, partials)


def kernel(pred_lines, confidence, targets):
    nsig = -jax.nn.sigmoid(confidence)
    pad_k = ((0, 0), (0, _KP - _K))
    p = pred_lines[..., :3]
    p0 = jnp.pad(p[..., 0], pad_k, constant_values=1e9)
    p1 = jnp.pad(p[..., 1], pad_k, constant_values=1e9)
    p2 = jnp.pad(p[..., 2], pad_k, constant_values=1e9)
    nsig_p = jnp.pad(nsig, pad_k)
    conf_p = jnp.pad(confidence, pad_k)
    pad_t = ((0, 0), (0, _TP - _T))
    t0 = jnp.pad(targets[..., 0], pad_t)
    t1 = jnp.pad(targets[..., 1], pad_t)
    t2 = jnp.pad(targets[..., 2], pad_t)
    partials = _sc_match(p0, p1, p2, nsig_p, conf_p, t0, t1, t2)
    out = _tc_combine(confidence, partials)
    return out[0, 0]


# speedup vs baseline: 10.2954x; 1.1379x over previous
import functools

import jax
import jax.numpy as jnp
from jax import lax
from jax.experimental import pallas as pl
from jax.experimental.pallas import tpu as pltpu
from jax.experimental.pallas import tpu_sc as plsc

_NC, _NS = 2, 16
_NW = _NC * _NS
_B, _K, _T = 64, 1000, 100
_KP, _TP = 1024, 128
_NCHUNK = 1008 // 16
_BPW = _B // _NW
_BIG = 1e30


def _rsqrt_newton(x):
    i = lax.bitcast_convert_type(x, jnp.int32)
    y = lax.bitcast_convert_type(
        jnp.int32(0x5F3759DF) - lax.shift_right_arithmetic(i, 1), jnp.float32)
    for _ in range(3):
        y = y * (1.5 - 0.5 * x * y * y)
    return y


def _sc_match(p0, p1, p2, nsig, conf, t0, t1, t2):
    mesh = plsc.VectorSubcoreMesh(core_axis_name="c", subcore_axis_name="s")
    fkp = lambda: pltpu.VMEM((_KP,), jnp.float32)
    ftp = lambda: pltpu.VMEM((_TP,), jnp.float32)

    @functools.partial(
        pl.kernel,
        out_type=jax.ShapeDtypeStruct((_B, 16), jnp.float32),
        mesh=mesh,
        scratch_types=[fkp(), fkp(), fkp(), fkp(), fkp(),
                       ftp(), ftp(), ftp(), ftp(), ftp(), ftp(),
                       pltpu.VMEM((16,), jnp.float32)],
        compiler_params=pltpu.CompilerParams(needs_layout_passes=False),
    )
    def k(p0_h, p1_h, p2_h, nsig_h, conf_h, t0_h, t1_h, t2_h, out_h,
          p0v, p1v, p2v, basev, confv, t0v, t1v, t2v, m0v, m1v, m2v, rowv):
        wid = lax.axis_index("s") * _NC + lax.axis_index("c")
        iota = lax.iota(jnp.int32, 16)
        for r in range(_BPW):
            b = wid * _BPW + r
            pltpu.sync_copy(p0_h.at[b], p0v)
            pltpu.sync_copy(p1_h.at[b], p1v)
            pltpu.sync_copy(p2_h.at[b], p2v)
            pltpu.sync_copy(nsig_h.at[b], basev)
            pltpu.sync_copy(conf_h.at[b], confv)
            pltpu.sync_copy(t0_h.at[b], t0v)
            pltpu.sync_copy(t1_h.at[b], t1v)
            pltpu.sync_copy(t2_h.at[b], t2v)

            lane0 = iota == 0

            def jstep(j, cm):
                j_s = jnp.full((16,), j, jnp.int32)
                t0s = plsc.load_gather(t0v, (j_s,))
                t1s = plsc.load_gather(t1v, (j_s,))
                t2s = plsc.load_gather(t2v, (j_s,))

                def cstep(c, carry):
                    rm, ri = carry
                    off = c * 16
                    v = (jnp.abs(p0v[pl.ds(off, 16)] - t0s)
                         + jnp.abs(p1v[pl.ds(off, 16)] - t1s)
                         + jnp.abs(p2v[pl.ds(off, 16)] - t2s)
                         ) + basev[pl.ds(off, 16)]
                    better = v < rm
                    rm = jnp.where(better, v, rm)
                    ri = jnp.where(better, off + iota, ri)
                    return rm, ri

                rm0 = jnp.full((16,), 3e38, jnp.float32)
                ri0 = jnp.zeros((16,), jnp.int32)
                rm, ri = lax.fori_loop(0, _NCHUNK, cstep, (rm0, ri0),
                                       unroll=9)
                mn = jnp.min(rm)
                idx = jnp.min(jnp.where(rm == mn, ri, jnp.int32(1 << 30)))
                idx_s = jnp.full((16,), idx, jnp.int32)
                plsc.store_scatter(basev, (idx_s,),
                                   jnp.full((16,), _BIG, jnp.float32),
                                   mask=lane0)
                plsc.store_scatter(m0v, (j_s,),
                                   plsc.load_gather(p0v, (idx_s,)), mask=lane0)
                plsc.store_scatter(m1v, (j_s,),
                                   plsc.load_gather(p1v, (idx_s,)), mask=lane0)
                plsc.store_scatter(m2v, (j_s,),
                                   plsc.load_gather(p2v, (idx_s,)), mask=lane0)
                return cm + plsc.load_gather(confv, (idx_s,))[0]

            cm = lax.fori_loop(0, _T, jstep, jnp.float32(0.0))

            dir_acc = jnp.zeros((16,), jnp.float32)
            off_acc = jnp.zeros((16,), jnp.float32)
            for cc in range((_T + 15) // 16):
                s = cc * 16
                a0 = m0v[pl.ds(s, 16)]
                a1 = m1v[pl.ds(s, 16)]
                ad = m2v[pl.ds(s, 16)]
                b0 = t0v[pl.ds(s, 16)]
                b1 = t1v[pl.ds(s, 16)]
                bd = t2v[pl.ds(s, 16)]
                sp = jnp.maximum(a0 * a0 + a1 * a1, 1e-24)
                st = jnp.maximum(b0 * b0 + b1 * b1, 1e-24)
                rp = _rsqrt_newton(sp)
                rt = _rsqrt_newton(st)
                u = (a0 * b0 + a1 * b1) * (rp * rt)
                dirt = jnp.abs(1.0 - u)
                offt = jnp.abs(ad * rp - bd * rt)
                msk = (s + iota) < _T
                dir_acc = dir_acc + jnp.where(msk, dirt, 0.0)
                off_acc = off_acc + jnp.where(msk, offt, 0.0)
            dir_s = jnp.sum(dir_acc)
            off_s = jnp.sum(off_acc)
            row = (jnp.where(iota == 0, dir_s, 0.0)
                   + jnp.where(iota == 1, off_s, 0.0)
                   + jnp.where(iota == 2, cm, 0.0))
            rowv[...] = row
            pltpu.sync_copy(rowv, out_h.at[b])

    return k(p0, p1, p2, nsig, conf, t0, t1, t2)


def _tc_combine(conf, partials):
    def body(conf_ref, part_ref, out_ref):
        x = conf_ref[...]
        bce = jnp.sum(jnp.maximum(x, 0.0) + jnp.log1p(jnp.exp(-jnp.abs(x))))
        pr = part_ref[...]
        dir_tot = jnp.sum(pr[:, 0:1])
        off_tot = jnp.sum(pr[:, 1:2])
        cm_tot = jnp.sum(pr[:, 2:3])
        inv_bt = 1.0 / (_B * _T)
        loss_lines = (dir_tot * inv_bt + off_tot * inv_bt) * 0.5
        loss_conf = (bce - cm_tot) * (1.0 / (_B * _K))
        out_ref[0, 0] = (loss_lines + loss_conf) * 0.5

    return pl.pallas_call(
        body,
        out_shape=jax.ShapeDtypeStruct((1, 1), jnp.float32),
        out_specs=pl.BlockSpec(memory_space=pltpu.SMEM),
    )(conf, partials)


def kernel(pred_lines, confidence, targets):
    nsig = -jax.nn.sigmoid(confidence)
    pad_k = ((0, 0), (0, _KP - _K))
    p = pred_lines[..., :3]
    p0 = jnp.pad(p[..., 0], pad_k, constant_values=1e9)
    p1 = jnp.pad(p[..., 1], pad_k, constant_values=1e9)
    p2 = jnp.pad(p[..., 2], pad_k, constant_values=1e9)
    nsig_p = jnp.pad(nsig, pad_k)
    conf_p = jnp.pad(confidence, pad_k)
    pad_t = ((0, 0), (0, _TP - _T))
    t0 = jnp.pad(targets[..., 0], pad_t)
    t1 = jnp.pad(targets[..., 1], pad_t)
    t2 = jnp.pad(targets[..., 2], pad_t)
    partials = _sc_match(p0, p1, p2, nsig_p, conf_p, t0, t1, t2)
    out = _tc_combine(confidence, partials)
    return out[0, 0]
